# baseline (speedup 1.0000x reference)
import functools

import jax
import jax.numpy as jnp
from jax import lax
from jax.experimental import pallas as pl
from jax.experimental.pallas import tpu as pltpu
from jax.experimental.pallas import tpu_sc as plsc

HIST = 50
DM = 64
GRP = 128
NBUF = 4
K = 3
NW = 32
LANES = 16


@jax.jit
def _sc_embedding_gather(w2, idx_g):
    nblocks = idx_g.shape[0]
    bpw = nblocks // NW
    mesh = plsc.VectorSubcoreMesh(core_axis_name="c", subcore_axis_name="s")
    nc = plsc.get_sparse_core_info().num_cores

    @functools.partial(
        pl.kernel,
        mesh=mesh,
        out_type=jax.ShapeDtypeStruct((HIST, 8, GRP, 8, GRP), jnp.float32),
        scratch_types=[
            pltpu.VMEM((bpw, GRP), jnp.int32),
            pltpu.VMEM((NBUF, GRP), jnp.int32),
            pltpu.VMEM((NBUF, GRP, GRP), jnp.float32),
            pltpu.VMEM((2, 8, 8, GRP), jnp.float32),
            pltpu.SemaphoreType.DMA((NBUF,)),
            pltpu.SemaphoreType.DMA((2,)),
        ],
        compiler_params=pltpu.CompilerParams(use_tc_tiling_on_sc=False,
                                             needs_layout_passes=False),
    )
    def k(w_hbm, idx_hbm, out_hbm, tok_v, sidx_v, rows_v, t_v, gsem, ssem):
        wid = lax.axis_index("s") * nc + lax.axis_index("c")
        base = wid * bpw
        pltpu.sync_copy(idx_hbm.at[pl.ds(base, bpw)], tok_v)

        def fire_gather(g, b):
            for c in range(GRP // LANES):
                tok = tok_v[g, pl.ds(c * LANES, LANES)]
                sidx_v[b, pl.ds(c * LANES, LANES)] = (
                    lax.shift_right_logical(tok, 1))
            pltpu.async_copy(w_hbm.at[sidx_v.at[b]], rows_v.at[b], gsem.at[b])

        def gather_wait(b):
            pltpu.make_async_copy(w_hbm.at[sidx_v.at[b]], rows_v.at[b],
                                  gsem.at[b]).wait()

        def scat_start(g, tb):
            blk = base + g
            h = blk // GRP
            bj = blk - h * GRP
            for ti in range(8):
                pltpu.async_copy(t_v.at[tb, ti], out_hbm.at[h, ti, bj],
                                 ssem.at[tb])

        def scat_wait(g, tb):
            blk = base + g
            h = blk // GRP
            bj = blk - h * GRP
            for ti in range(8):
                pltpu.make_async_copy(t_v.at[tb, ti], out_hbm.at[h, ti, bj],
                                      ssem.at[tb]).wait()

        def transpose(g, b, tb):
            row0 = lax.iota(jnp.int32, LANES)
            sel64s, rowss = [], []
            for c in range(GRP // LANES):
                tok = tok_v[g, pl.ds(c * LANES, LANES)]
                sel64s.append(lax.shift_left(jnp.bitwise_and(tok, 1), 6))
                rowss.append(row0 + (c * LANES))

            def dloop(d, _):
                ti = lax.shift_right_logical(d, 3)
                s = d - ti * 8
                for c in range(GRP // LANES):
                    vals = plsc.load_gather(rows_v.at[b],
                                            [rowss[c], sel64s[c] + d])
                    t_v[tb, ti, s, pl.ds(c * LANES, LANES)] = vals
                return 0

            lax.fori_loop(0, DM, dloop, 0)

        for g0 in range(K):
            fire_gather(g0, g0)

        def outer(t, _):
            for j in range(NBUF):
                g = t * NBUF + j
                tb = j % 2

                @pl.when(g >= 2)
                def _():
                    scat_wait(g - 2, tb)

                gather_wait(j)
                transpose(g, j, tb)
                scat_start(g, tb)

                @pl.when(g + K < bpw)
                def _():
                    fire_gather(g + K, (j + K) % NBUF)

            return 0

        lax.fori_loop(0, bpw // NBUF, outer, 0)
        for g in range(bpw - 2, bpw):
            scat_wait(g, g % 2)

    return k(w2, idx_g)


def kernel(token_ids, weight):
    bsz, h = token_ids.shape
    idx_g = token_ids.astype(jnp.int32).T.reshape(h * (bsz // GRP), GRP)
    w2 = weight.reshape(weight.shape[0] // 2, 2 * weight.shape[1])
    out5 = _sc_embedding_gather(w2, idx_g)
    return out5.transpose(2, 4, 0, 1, 3).reshape(bsz, h, DM)
